# Initial kernel scaffold; baseline (speedup 1.0000x reference)
import jax
import jax.numpy as jnp
from jax.experimental import pallas as pl


def kernel(arsenal_z, pitch_mix, hitter_idx, intercept, beta_arsenal, beta_ptmix, hitter_base, hitter_arsenal, hitter_ptype):
    raise NotImplementedError("write your pallas kernel here")


# speedup vs baseline: 2.2261x; 2.2261x over previous
import functools

import jax
import jax.numpy as jnp
from jax import lax
from jax.experimental import pallas as pl
from jax.experimental.pallas import tpu as pltpu
from jax.experimental.pallas import tpu_sc as plsc

_A = 16
_P = 11
_L = 10
_LANES = 16
_BA_PAD = _A * _L + _LANES
_BP_PAD = _P * _L + _LANES + 2


def _matchup_body(rows_per_w, chunk, num_cores, maxb, maxp,
                  az_hbm, pm_hbm, idx_hbm, ic_hbm, ba_hbm, bp_hbm,
                  hb16_hbm, hars_hbm, pt16_hbm, out_hbm,
                  idx_v, idx2_v, idx8_v, remb_v, remp_v,
                  az_v, pm_v, hb2_v, ars_v, pt8_v, out_v,
                  ic_vv, ba_vv, bp_vv, ic_s, ba_s, bp_s, sem0, sem1, sem2):
    wid = lax.axis_index("s") * num_cores + lax.axis_index("c")
    base_row = wid * rows_per_w
    pltpu.sync_copy(ic_hbm, ic_vv)
    pltpu.sync_copy(ba_hbm, ba_vv)
    pltpu.sync_copy(bp_hbm, bp_vv)
    v = ic_vv[pl.ds(0, _LANES)]
    for j in range(_L):
        ic_s[j] = v[j]
    for a in range(_A):
        va = ba_vv[pl.ds(a * _L, _LANES)]
        for j in range(_L):
            ba_s[a * _L + j] = va[j]
    for p in range(_P):
        vp = bp_vv[pl.ds(p * _L, _LANES)]
        for j in range(_L):
            bp_s[p * _L + j] = vp[j]

    n_groups = chunk // _LANES
    iota = lax.iota(jnp.int32, _LANES)

    def splat(val):
        return jnp.full((_LANES,), val, jnp.int32)

    for c in range(rows_per_w // chunk):
        off = base_row + c * chunk
        pltpu.sync_copy(idx_hbm.at[pl.ds(off, chunk)], idx_v)

        def build(g, carry):
            rows = iota + g * _LANES
            hidx = idx_v[pl.ds(g * _LANES, _LANES)]
            tb = hidx * _L
            g0b = tb >> 4
            remb_v[pl.ds(g * _LANES, _LANES)] = tb - g0b * _LANES
            rows2 = rows * 2
            plsc.store_scatter(idx2_v, [rows2], g0b)
            plsc.store_scatter(idx2_v, [rows2 + 1], jnp.minimum(g0b + 1, maxb))
            tp = hidx * (_P * _L)
            g0p = tp >> 4
            remp_v[pl.ds(g * _LANES, _LANES)] = tp - g0p * _LANES
            rows8 = rows * 8
            plsc.store_scatter(idx8_v, [rows8], g0p)
            for k in range(1, 8):
                plsc.store_scatter(idx8_v, [rows8 + k],
                                   jnp.minimum(g0p + k, maxp))
            return carry

        lax.fori_loop(0, n_groups, build, 0)

        cp0 = pltpu.async_copy(hb16_hbm.at[idx2_v], hb2_v, sem0)
        cp1 = pltpu.async_copy(hars_hbm.at[idx_v], ars_v, sem1)
        cp2 = pltpu.async_copy(pt16_hbm.at[idx8_v], pt8_v, sem2)
        pltpu.sync_copy(az_hbm.at[pl.ds(off, chunk), :], az_v)
        pltpu.sync_copy(pm_hbm.at[pl.ds(off, chunk), :], pm_v)
        cp0.wait()
        cp1.wait()
        cp2.wait()

        def group(g, carry):
            rows = iota + g * _LANES
            rows2 = rows * 2
            rows8 = rows * 8
            remb = remb_v[pl.ds(g * _LANES, _LANES)]
            remp = remp_v[pl.ds(g * _LANES, _LANES)]
            zv = [plsc.load_gather(az_v, [rows, splat(a)]) for a in range(_A)]
            mv = [plsc.load_gather(pm_v, [rows, splat(p)]) for p in range(_P)]
            acc = []
            for j in range(_L):
                t = remb + j
                hb = plsc.load_gather(hb2_v, [rows2 + (t >> 4), t & 15])
                acc.append(hb + ic_s[j])
            for a in range(_A):
                for j in range(_L):
                    h = plsc.load_gather(ars_v, [rows, splat(a * _L + j)])
                    acc[j] = acc[j] + zv[a] * (h + ba_s[a * _L + j])
            for p in range(_P):
                for j in range(_L):
                    t = remp + (p * _L + j)
                    h = plsc.load_gather(pt8_v, [rows8 + (t >> 4), t & 15])
                    acc[j] = acc[j] + mv[p] * (h + bp_s[p * _L + j])
            m = acc[0]
            for j in range(1, _L):
                m = jnp.maximum(m, acc[j])
            m = jnp.maximum(m, jnp.float32(0.0))
            e = [jnp.exp(acc[j] - m) for j in range(_L)]
            ez = jnp.exp(-m)
            s = ez
            for j in range(_L):
                s = s + e[j]
            inv = jnp.float32(1.0) / s
            for j in range(_L):
                plsc.store_scatter(out_v, [rows, splat(j)], e[j] * inv)
            plsc.store_scatter(out_v, [rows, splat(_L)], ez * inv)
            return carry

        lax.fori_loop(0, n_groups, group, 0)
        pltpu.sync_copy(out_v, out_hbm.at[pl.ds(off, chunk), :])


def kernel(arsenal_z, pitch_mix, hitter_idx, intercept, beta_arsenal,
           beta_ptmix, hitter_base, hitter_arsenal, hitter_ptype):
    batch = arsenal_z.shape[0]
    n_hitters = hitter_base.shape[0]
    idx = hitter_idx.astype(jnp.int32)
    hb16 = hitter_base.reshape(n_hitters * _L // _LANES, _LANES)
    hars2 = hitter_arsenal.reshape(n_hitters, _A * _L)
    pt16 = hitter_ptype.reshape(n_hitters * _P * _L // _LANES, _LANES)
    ic16 = jnp.zeros((_LANES,), jnp.float32).at[:_L].set(intercept)
    ba_flat = jnp.zeros((_BA_PAD,), jnp.float32).at[:_A * _L].set(
        beta_arsenal.ravel())
    bp_flat = jnp.zeros((_BP_PAD,), jnp.float32).at[:_P * _L].set(
        beta_ptmix.ravel())

    info = plsc.get_sparse_core_info()
    num_workers = info.num_cores * info.num_subcores
    rows_per_w = batch // num_workers
    chunk = 128
    assert batch % (num_workers * chunk) == 0

    mesh = plsc.VectorSubcoreMesh(core_axis_name="c", subcore_axis_name="s")
    body = functools.partial(
        _matchup_body, rows_per_w, chunk, info.num_cores,
        jnp.int32(n_hitters * _L // _LANES - 1),
        jnp.int32(n_hitters * _P * _L // _LANES - 1))
    probs = pl.kernel(
        body,
        out_type=jax.ShapeDtypeStruct((batch, _L + 1), jnp.float32),
        mesh=mesh,
        compiler_params=pltpu.CompilerParams(
            needs_layout_passes=False, use_tc_tiling_on_sc=False),
        scratch_types=[
            pltpu.VMEM((chunk,), jnp.int32),
            pltpu.VMEM((2 * chunk,), jnp.int32),
            pltpu.VMEM((8 * chunk,), jnp.int32),
            pltpu.VMEM((chunk,), jnp.int32),
            pltpu.VMEM((chunk,), jnp.int32),
            pltpu.VMEM((chunk, _A), jnp.float32),
            pltpu.VMEM((chunk, _P), jnp.float32),
            pltpu.VMEM((2 * chunk, _LANES), jnp.float32),
            pltpu.VMEM((chunk, _A * _L), jnp.float32),
            pltpu.VMEM((8 * chunk, _LANES), jnp.float32),
            pltpu.VMEM((chunk, _L + 1), jnp.float32),
            pltpu.VMEM((_LANES,), jnp.float32),
            pltpu.VMEM((_BA_PAD,), jnp.float32),
            pltpu.VMEM((_BP_PAD,), jnp.float32),
            pltpu.SMEM((_L,), jnp.float32),
            pltpu.SMEM((_A * _L,), jnp.float32),
            pltpu.SMEM((_P * _L,), jnp.float32),
            pltpu.SemaphoreType.DMA,
            pltpu.SemaphoreType.DMA,
            pltpu.SemaphoreType.DMA,
        ],
    )(arsenal_z, pitch_mix, idx, ic16, ba_flat, bp_flat,
      hb16, hars2, pt16)
    return probs
